# baseline (speedup 1.0000x reference)
import jax
import jax.numpy as jnp
from jax import lax
from jax.experimental import pallas as pl
from jax.experimental.pallas import tpu as pltpu
from jax.experimental.pallas import tpu_sc as plsc

NG = 64
STOP = 20.0
E = 800000
C = 250
NCHUNK = E // C
NW = 32
KPW = NCHUNK // NW
NT = -(-C // 16)
NGP = NG + 1

_STEP = STOP / (NG - 1)
_COEFF = -0.5 / _STEP ** 2
_OFFS = [i * _STEP for i in range(NG)]
_SC = (-_COEFF) ** 0.5
_SOFFS = [o * _SC for o in _OFFS]


def _sc_body(el_hbm, et_hbm, emb_hbm, out_hbm,
             len_v, idx_v, rbf_v, bond_v, isem, osem, gsem):
    cid = lax.axis_index("c")
    sid = lax.axis_index("s")
    wid = sid * 2 + cid
    start = wid * KPW

    def issue_inputs(k, s):
        pltpu.async_copy(el_hbm.at[start + k], len_v[s].at[pl.ds(0, C)],
                         isem[s])
        pltpu.async_copy(et_hbm.at[start + k], idx_v[s], isem[s])

    def wait_inputs(k, s):
        pltpu.make_async_copy(el_hbm.at[start + k],
                              len_v[s].at[pl.ds(0, C)], isem[s]).wait()
        pltpu.make_async_copy(et_hbm.at[start + k], idx_v[s],
                              isem[s]).wait()

    def out_slices(k):
        rows = pl.ds((start + k) * C, C)
        return out_hbm.at[rows, pl.ds(0, NG)], out_hbm.at[rows, pl.ds(NG, NG)]

    def process(k, s):
        o_rbf, o_bond = out_slices(k)

        @pl.when(k >= 2)
        def _():
            pltpu.make_async_copy(rbf_v[s].at[:, pl.ds(0, NG)], o_rbf, osem[s]).wait()
            pltpu.make_async_copy(bond_v[s], o_bond, osem[s]).wait()

        wait_inputs(k, s)
        g0 = pltpu.async_copy(emb_hbm.at[idx_v[s].at[0]],
                              bond_v[s].at[pl.ds(0, 125)], gsem)
        g1 = pltpu.async_copy(emb_hbm.at[idx_v[s].at[1]],
                              bond_v[s].at[pl.ds(125, 125)], gsem)

        @pl.when(k + 1 < KPW)
        def _():
            issue_inputs(k + 1, 1 - s)

        def t_body(t, _):
            u = len_v[s][pl.ds(t * 16, 16)] * _SC
            rows = t * 16 + lax.iota(jnp.int32, 16)
            mask = rows < C
            for j in range(NG):
                v = jnp.exp((u - _SOFFS[j]) * (_SOFFS[j] - u))
                plsc.store_scatter(
                    rbf_v[s], [rows, jnp.full((16,), j, jnp.int32)], v,
                    mask=mask)
            return 0

        lax.fori_loop(0, NT, t_body, 0)
        g0.wait()
        g1.wait()
        pltpu.async_copy(rbf_v[s].at[:, pl.ds(0, NG)], o_rbf, osem[s])
        pltpu.async_copy(bond_v[s], o_bond, osem[s])

    issue_inputs(0, 0)

    def pair_body(kk, carry):
        process(kk * 2, 0)
        process(kk * 2 + 1, 1)
        return carry

    lax.fori_loop(0, KPW // 2, pair_body, 0)

    for k, s in ((KPW - 2, 0), (KPW - 1, 1)):
        o_rbf, o_bond = out_slices(k)
        pltpu.make_async_copy(rbf_v[s].at[:, pl.ds(0, NG)], o_rbf, osem[s]).wait()
        pltpu.make_async_copy(bond_v[s], o_bond, osem[s]).wait()


def kernel(edge_length, edge_type, emb_table):
    el = edge_length.reshape(NCHUNK, C)
    et = edge_type.astype(jnp.int32).reshape(NCHUNK, 2, 125)

    f = pl.kernel(
        _sc_body,
        mesh=plsc.VectorSubcoreMesh(core_axis_name="c", subcore_axis_name="s"),
        out_type=jax.ShapeDtypeStruct((E, 2 * NG), jnp.float32),
        scratch_types=[
            [pltpu.VMEM((256,), jnp.float32) for _ in range(2)],
            [pltpu.VMEM((2, 125), jnp.int32) for _ in range(2)],
            [pltpu.VMEM((C, NGP), jnp.float32) for _ in range(2)],
            [pltpu.VMEM((C, NG), jnp.float32) for _ in range(2)],
            [pltpu.SemaphoreType.DMA for _ in range(2)],
            [pltpu.SemaphoreType.DMA for _ in range(2)],
            pltpu.SemaphoreType.DMA,
        ],
        compiler_params=pltpu.CompilerParams(
            use_tc_tiling_on_sc=False, needs_layout_passes=False),
    )
    return f(el, et, emb_table)


# speedup vs baseline: 1.0792x; 1.0792x over previous
import jax
import jax.numpy as jnp
from jax import lax
from jax.experimental import pallas as pl
from jax.experimental.pallas import tpu as pltpu
from jax.experimental.pallas import tpu_sc as plsc

NG = 64
STOP = 20.0
E = 800000
C = 250
NCHUNK = E // C
NW = 32
KPW = NCHUNK // NW
NT = -(-C // 16)

_STEP = STOP / (NG - 1)
_COEFF = -0.5 / _STEP ** 2
_OFFS = [i * _STEP for i in range(NG)]
_SC = (-_COEFF) ** 0.5
_SOFFS = [o * _SC for o in _OFFS]


def _sc_body(el_hbm, et_hbm, emb_hbm, out_hbm,
             len_v, idx_v, rbf_v, bond_v, isem, osem, gsem):
    cid = lax.axis_index("c")
    sid = lax.axis_index("s")
    wid = sid * 2 + cid
    start = wid * KPW

    def issue_inputs(k, s):
        pltpu.async_copy(el_hbm.at[start + k], len_v[s].at[pl.ds(0, C)],
                         isem[s])
        pltpu.async_copy(et_hbm.at[start + k], idx_v[s], isem[s])

    def wait_inputs(k, s):
        pltpu.make_async_copy(el_hbm.at[start + k],
                              len_v[s].at[pl.ds(0, C)], isem[s]).wait()
        pltpu.make_async_copy(et_hbm.at[start + k], idx_v[s],
                              isem[s]).wait()

    def out_slices(k):
        rows = pl.ds((start + k) * C, C)
        return out_hbm.at[rows, pl.ds(0, NG)], out_hbm.at[rows, pl.ds(NG, NG)]

    def process(k, s):
        o_rbf, o_bond = out_slices(k)

        @pl.when(k >= 2)
        def _():
            pltpu.make_async_copy(rbf_v[s], o_rbf, osem[s]).wait()
            pltpu.make_async_copy(bond_v[s], o_bond, osem[s]).wait()

        wait_inputs(k, s)
        g0 = pltpu.async_copy(emb_hbm.at[idx_v[s].at[0]],
                              bond_v[s].at[pl.ds(0, 125)], gsem)
        g1 = pltpu.async_copy(emb_hbm.at[idx_v[s].at[1]],
                              bond_v[s].at[pl.ds(125, 125)], gsem)

        @pl.when(k + 1 < KPW)
        def _():
            issue_inputs(k + 1, 1 - s)

        def t_body(t, _):
            u = len_v[s][pl.ds(t * 16, 16)] * _SC
            rows = t * 16 + lax.iota(jnp.int32, 16)
            mask = rows < C
            for j in range(NG):
                v = jnp.exp((u - _SOFFS[j]) * (_SOFFS[j] - u))
                plsc.store_scatter(
                    rbf_v[s], [rows, jnp.full((16,), j, jnp.int32)], v,
                    mask=mask)
            return 0

        lax.fori_loop(0, NT, t_body, 0)
        g0.wait()
        g1.wait()
        pltpu.async_copy(rbf_v[s], o_rbf, osem[s])
        pltpu.async_copy(bond_v[s], o_bond, osem[s])

    issue_inputs(0, 0)

    def pair_body(kk, carry):
        process(kk * 2, 0)
        process(kk * 2 + 1, 1)
        return carry

    lax.fori_loop(0, KPW // 2, pair_body, 0)

    for k, s in ((KPW - 2, 0), (KPW - 1, 1)):
        o_rbf, o_bond = out_slices(k)
        pltpu.make_async_copy(rbf_v[s], o_rbf, osem[s]).wait()
        pltpu.make_async_copy(bond_v[s], o_bond, osem[s]).wait()


def kernel(edge_length, edge_type, emb_table):
    el = edge_length.reshape(NCHUNK, C)
    et = edge_type.astype(jnp.int32).reshape(NCHUNK, 2, 125)

    f = pl.kernel(
        _sc_body,
        mesh=plsc.VectorSubcoreMesh(core_axis_name="c", subcore_axis_name="s"),
        out_type=jax.ShapeDtypeStruct((E, 2 * NG), jnp.float32),
        scratch_types=[
            [pltpu.VMEM((256,), jnp.float32) for _ in range(2)],
            [pltpu.VMEM((2, 125), jnp.int32) for _ in range(2)],
            [pltpu.VMEM((C, NG), jnp.float32) for _ in range(2)],
            [pltpu.VMEM((C, NG), jnp.float32) for _ in range(2)],
            [pltpu.SemaphoreType.DMA for _ in range(2)],
            [pltpu.SemaphoreType.DMA for _ in range(2)],
            pltpu.SemaphoreType.DMA,
        ],
        compiler_params=pltpu.CompilerParams(
            use_tc_tiling_on_sc=False, needs_layout_passes=False),
    )
    return f(el, et, emb_table)


# speedup vs baseline: 1.0857x; 1.0060x over previous
import jax
import jax.numpy as jnp
from jax import lax
from jax.experimental import pallas as pl
from jax.experimental.pallas import tpu as pltpu
from jax.experimental.pallas import tpu_sc as plsc

NG = 64
STOP = 20.0
E = 800000
C = 250
NCHUNK = E // C
NW = 32
KPW = NCHUNK // NW
NT = -(-C // 16)

_STEP = STOP / (NG - 1)
_COEFF = -0.5 / _STEP ** 2
_OFFS = [i * _STEP for i in range(NG)]


def _sc_body(el_hbm, et_hbm, emb_hbm, out_hbm,
             len_v, idx_v, rbf_v, bond_v, isem, osem, gsem):
    cid = lax.axis_index("c")
    sid = lax.axis_index("s")
    wid = sid * 2 + cid
    start = wid * KPW

    def issue_inputs(k, s):
        pltpu.async_copy(el_hbm.at[start + k], len_v[s].at[pl.ds(0, C)],
                         isem[s])
        pltpu.async_copy(et_hbm.at[start + k], idx_v[s], isem[s])

    def wait_inputs(k, s):
        pltpu.make_async_copy(el_hbm.at[start + k],
                              len_v[s].at[pl.ds(0, C)], isem[s]).wait()
        pltpu.make_async_copy(et_hbm.at[start + k], idx_v[s],
                              isem[s]).wait()

    def out_slices(k):
        rows = pl.ds((start + k) * C, C)
        return out_hbm.at[rows, pl.ds(0, NG)], out_hbm.at[rows, pl.ds(NG, NG)]

    def process(k, s):
        o_rbf, o_bond = out_slices(k)

        @pl.when(k >= 2)
        def _():
            pltpu.make_async_copy(rbf_v[s], o_rbf, osem[s]).wait()
            pltpu.make_async_copy(bond_v[s], o_bond, osem[s]).wait()

        wait_inputs(k, s)
        g0 = pltpu.async_copy(emb_hbm.at[idx_v[s].at[0]],
                              bond_v[s].at[pl.ds(0, 125)], gsem)
        g1 = pltpu.async_copy(emb_hbm.at[idx_v[s].at[1]],
                              bond_v[s].at[pl.ds(125, 125)], gsem)

        @pl.when(k + 1 < KPW)
        def _():
            issue_inputs(k + 1, 1 - s)

        def t_body(t, _):
            lv = len_v[s][pl.ds(t * 16, 16)]
            rows = t * 16 + lax.iota(jnp.int32, 16)
            mask = rows < C
            for j in range(NG):
                d = lv - _OFFS[j]
                v = jnp.exp((_COEFF * d) * d)
                plsc.store_scatter(
                    rbf_v[s], [rows, jnp.full((16,), j, jnp.int32)], v,
                    mask=mask)
            return 0

        lax.fori_loop(0, NT, t_body, 0)
        g0.wait()
        g1.wait()
        pltpu.async_copy(rbf_v[s], o_rbf, osem[s])
        pltpu.async_copy(bond_v[s], o_bond, osem[s])

    issue_inputs(0, 0)

    def pair_body(kk, carry):
        process(kk * 2, 0)
        process(kk * 2 + 1, 1)
        return carry

    lax.fori_loop(0, KPW // 2, pair_body, 0)

    for k, s in ((KPW - 2, 0), (KPW - 1, 1)):
        o_rbf, o_bond = out_slices(k)
        pltpu.make_async_copy(rbf_v[s], o_rbf, osem[s]).wait()
        pltpu.make_async_copy(bond_v[s], o_bond, osem[s]).wait()


def kernel(edge_length, edge_type, emb_table):
    el = edge_length.reshape(NCHUNK, C)
    et = edge_type.astype(jnp.int32).reshape(NCHUNK, 2, 125)

    f = pl.kernel(
        _sc_body,
        mesh=plsc.VectorSubcoreMesh(core_axis_name="c", subcore_axis_name="s"),
        out_type=jax.ShapeDtypeStruct((E, 2 * NG), jnp.float32),
        scratch_types=[
            [pltpu.VMEM((256,), jnp.float32) for _ in range(2)],
            [pltpu.VMEM((2, 125), jnp.int32) for _ in range(2)],
            [pltpu.VMEM((C, NG), jnp.float32) for _ in range(2)],
            [pltpu.VMEM((C, NG), jnp.float32) for _ in range(2)],
            [pltpu.SemaphoreType.DMA for _ in range(2)],
            [pltpu.SemaphoreType.DMA for _ in range(2)],
            pltpu.SemaphoreType.DMA,
        ],
        compiler_params=pltpu.CompilerParams(
            use_tc_tiling_on_sc=False, needs_layout_passes=False),
    )
    return f(el, et, emb_table)
